# Initial kernel scaffold; baseline (speedup 1.0000x reference)
import jax
import jax.numpy as jnp
from jax.experimental import pallas as pl


def kernel(padded_bags, key_padding_mask, input_ids, attention_mask, text_emb_table, text_pool_W, text_pool_b, text_proj_W, text_proj_b, enc_W, enc_b, ln_g, ln_b, Wq, bq, Wk, bk, Wv, bv, Wo, bo, cls_W, cls_b):
    raise NotImplementedError("write your pallas kernel here")


# speedup vs baseline: 1.2338x; 1.2338x over previous
import functools

import jax
import jax.numpy as jnp
from jax import lax
from jax.experimental import pallas as pl
from jax.experimental.pallas import tpu as pltpu
from jax.experimental.pallas import tpu_sc as plsc

THR = 0.7
GAMMA = 0.8
L_MAX = 1024
TN = 2048


def _sc_gather(table, idx, rows_per_chunk):
  M = idx.shape[0]
  D = table.shape[1]
  info = plsc.get_sparse_core_info()
  nw = info.num_cores * info.num_subcores
  b_per_w = M // nw
  nchunks = b_per_w // rows_per_chunk
  mesh = plsc.VectorSubcoreMesh(core_axis_name="c", subcore_axis_name="s")

  @functools.partial(
      pl.kernel,
      mesh=mesh,
      out_type=jax.ShapeDtypeStruct((M, D), table.dtype),
      scratch_types=[
          pltpu.VMEM((rows_per_chunk,), jnp.int32),
          pltpu.VMEM((rows_per_chunk, D), table.dtype),
          pltpu.SemaphoreType.DMA,
      ],
  )
  def gk(table_hbm, idx_hbm, out_hbm, idx_v, rows_v, sem):
    wid = lax.axis_index("s") * info.num_cores + lax.axis_index("c")
    base = wid * b_per_w
    for c in range(nchunks):
      off = base + c * rows_per_chunk
      pltpu.sync_copy(idx_hbm.at[pl.ds(off, rows_per_chunk)], idx_v)
      pltpu.async_copy(table_hbm.at[idx_v], rows_v, sem).wait()
      pltpu.sync_copy(rows_v, out_hbm.at[pl.ds(off, rows_per_chunk)])

  return gk(table, idx)


def _text_body(tok_ref, am_ref, pw_ref, pb_ref, qw_ref, qb_ref, out_ref):
  tok = tok_ref[...]
  m = am_ref[...]
  pooled = (tok * m[..., None]).sum(1) / jnp.maximum(m.sum(1, keepdims=True), 1.0)
  feat = jnp.tanh(jnp.dot(pooled, pw_ref[...],
                          preferred_element_type=jnp.float32) + pb_ref[...])
  out_ref[...] = jnp.dot(feat, qw_ref[...],
                         preferred_element_type=jnp.float32) + qb_ref[...]


def _text_emb(tok, am_f, pool_W, pool_b, proj_W, proj_b):
  B = tok.shape[0]
  D = proj_W.shape[1]
  return pl.pallas_call(
      _text_body,
      out_shape=jax.ShapeDtypeStruct((B, D), jnp.float32),
  )(tok, am_f, pool_W, pool_b.reshape(1, -1), proj_W, proj_b.reshape(1, -1))


def _enc_tile(x, w, b, g, bb):
  h = jnp.dot(x, w, preferred_element_type=jnp.float32) + b
  mu = h.mean(-1, keepdims=True)
  var = ((h - mu) ** 2).mean(-1, keepdims=True)
  h = (h - mu) / jnp.sqrt(var + 1e-5) * g + bb
  return jnp.maximum(h, 0.0)


def _enc_body(bags_ref, kpm_ref, temb_ref, w_ref, b_ref, g_ref, bb_ref,
              rel_ref, val_ref, carry_ref):
  j = pl.program_id(1)
  img = _enc_tile(bags_ref[0], w_ref[...], b_ref[...], g_ref[...], bb_ref[...])
  t = temb_ref[0]
  rel = jnp.dot(img, t.reshape(-1, 1),
                preferred_element_type=jnp.float32)
  nrm = img / (jnp.sqrt((img * img).sum(-1, keepdims=True)) + 1e-8)
  prev = jnp.concatenate([carry_ref[0:1], nrm[:-1]], axis=0)
  sim = (nrm * prev).sum(-1, keepdims=True)
  carry_ref[0:1] = nrm[-1:]
  red = (sim > THR).astype(jnp.float32)
  row = lax.broadcasted_iota(jnp.int32, sim.shape, 0)
  red = jnp.where((j == 0) & (row == 0), 0.0, red)
  valid = (1.0 - kpm_ref[0]) * (1.0 - red)
  rel_ref[0] = jnp.where(valid > 0.0, rel, -1e9)
  val_ref[0] = valid


def _encode_rel(bags, kpm_f, temb3, enc_W, enc_b, ln_g, ln_b):
  B, N, FD = bags.shape
  D = enc_W.shape[1]
  nt = N // TN
  grid = (B, nt)
  out = pl.pallas_call(
      _enc_body,
      grid=grid,
      in_specs=[
          pl.BlockSpec((1, TN, FD), lambda b, j: (b, j, 0)),
          pl.BlockSpec((1, TN, 1), lambda b, j: (b * nt + j, 0, 0)),
          pl.BlockSpec((1, 1, D), lambda b, j: (b, 0, 0)),
          pl.BlockSpec((FD, D), lambda b, j: (0, 0)),
          pl.BlockSpec((1, D), lambda b, j: (0, 0)),
          pl.BlockSpec((1, D), lambda b, j: (0, 0)),
          pl.BlockSpec((1, D), lambda b, j: (0, 0)),
      ],
      out_specs=[
          pl.BlockSpec((1, TN, 1), lambda b, j: (b * nt + j, 0, 0)),
          pl.BlockSpec((1, TN, 1), lambda b, j: (b * nt + j, 0, 0)),
      ],
      out_shape=[
          jax.ShapeDtypeStruct((B * nt, TN, 1), jnp.float32),
          jax.ShapeDtypeStruct((B * nt, TN, 1), jnp.float32),
      ],
      scratch_shapes=[pltpu.VMEM((8, D), jnp.float32)],
      compiler_params=pltpu.CompilerParams(
          dimension_semantics=("arbitrary", "arbitrary")),
  )(bags, kpm_f.reshape(B * nt, TN, 1), temb3,
    enc_W, enc_b.reshape(1, D), ln_g.reshape(1, D), ln_b.reshape(1, D))
  return out[0].reshape(B, N), out[1].reshape(B, N)


def _sel_body(kk, N, rel_ref, val_ref, gidx0_ref, sv0_ref, gidx1_ref, sv1_ref):
  b = pl.program_id(0)
  x = rel_ref[0]
  R, C = x.shape
  bits = lax.bitcast_convert_type(x, jnp.int32)
  key = jnp.where(bits >= 0, bits, bits ^ jnp.int32(2147483647))

  cnt_pos = jnp.sum((key >= 0).astype(jnp.int32))
  t0 = jnp.where(cnt_pos >= kk, jnp.int32(0), jnp.iinfo(jnp.int32).min)

  def bit_step(i, t):
    cand = t | (jnp.int32(1) << (jnp.int32(30) - i))
    cnt = jnp.sum((key >= cand).astype(jnp.int32))
    return jnp.where(cnt >= kk, cand, t)

  t = lax.fori_loop(0, 31, bit_step, t0)

  gt = (key > t).astype(jnp.float32)
  eq = (key == t).astype(jnp.float32)
  need = kk - jnp.sum(gt)

  hi = lax.Precision.HIGHEST
  ri = lax.broadcasted_iota(jnp.int32, (R, C), 0)
  ci = lax.broadcasted_iota(jnp.int32, (R, C), 1)
  m_inc = (ri <= ci).astype(jnp.float32)
  m_gt = (ri > ci).astype(jnp.float32)
  m_ge = (ri >= ci).astype(jnp.float32)

  incl_eq = jnp.dot(eq, m_inc, precision=hi)
  rowsum_eq = incl_eq[:, C - 1:C]
  rowpref_eq = jnp.dot(m_gt, rowsum_eq, precision=hi)
  excl_eq = rowpref_eq + incl_eq - eq
  take = gt + eq * (excl_eq < need).astype(jnp.float32)

  incl_tk = jnp.dot(take, m_inc, precision=hi)
  rowsum_tk = incl_tk[:, C - 1:C]
  rowincl_tk = jnp.dot(m_ge, rowsum_tk, precision=hi)
  rowpref_tk = rowincl_tk - rowsum_tk

  half = kk // 2

  def extract(off):
    jj = (lax.broadcasted_iota(jnp.int32, (half, C), 0) * 2
          + off).astype(jnp.float32)
    cc = lax.broadcasted_iota(jnp.int32, (half, C), 1).astype(jnp.float32)
    r_j = (rowincl_tk.reshape(1, C) <= jj).astype(jnp.float32).sum(
        -1, keepdims=True)
    onehot_r = (cc == r_j).astype(jnp.float32)
    rowpref_j = jnp.dot(onehot_r, rowpref_tk, precision=hi)
    jl = jj[:, 0:1] - rowpref_j
    inclrows = jnp.dot(onehot_r, incl_tk, precision=hi)
    c_j = (inclrows <= jl).astype(jnp.float32).sum(-1, keepdims=True)
    onehot_c = (cc == c_j).astype(jnp.float32)
    vrows = jnp.dot(onehot_r, val_ref[0], precision=hi)
    sv = (vrows * onehot_c).sum(-1, keepdims=True)
    gidx = (r_j * C + c_j).astype(jnp.int32) + b * N
    return gidx, sv

  g0, s0 = extract(0)
  g1, s1 = extract(1)
  gidx0_ref[0] = g0
  sv0_ref[0] = s0
  gidx1_ref[0] = g1
  sv1_ref[0] = s1


def _select(rel, val, kk):
  B, N = rel.shape
  R = 128
  C = N // R
  half = kk // 2
  ospec = pl.BlockSpec((1, half, 1), lambda b: (b, 0, 0))
  return pl.pallas_call(
      functools.partial(_sel_body, kk, N),
      grid=(B,),
      in_specs=[
          pl.BlockSpec((1, R, C), lambda b: (b, 0, 0)),
          pl.BlockSpec((1, R, C), lambda b: (b, 0, 0)),
      ],
      out_specs=[ospec, ospec, ospec, ospec],
      out_shape=[
          jax.ShapeDtypeStruct((B, half, 1), jnp.int32),
          jax.ShapeDtypeStruct((B, half, 1), jnp.float32),
          jax.ShapeDtypeStruct((B, half, 1), jnp.int32),
          jax.ShapeDtypeStruct((B, half, 1), jnp.float32),
      ],
  )(rel.reshape(B, R, C), val.reshape(B, R, C))


def _tail_body(H, rows0_ref, rows1_ref, sv0_ref, sv1_ref, temb_ref,
               w_ref, b_ref, g_ref, bb_ref,
               wq_ref, bq_ref, wk_ref, bk_ref, wv_ref, bv_ref,
               wo_ref, bo_ref, cw_ref, cb_ref, out_ref):
  f0 = _enc_tile(rows0_ref[0], w_ref[...], b_ref[...], g_ref[...],
                 bb_ref[...])
  f1 = _enc_tile(rows1_ref[0], w_ref[...], b_ref[...], g_ref[...],
                 bb_ref[...])
  L, D = f0.shape
  v0 = sv0_ref[0]
  v1 = sv1_ref[0]
  comp = (f0 * v0 + f1 * v1) / jnp.maximum(v0 + v1, 1.0)
  cv = (v0 + v1) > 0.0

  t = temb_ref[0]
  q = jnp.dot(t, wq_ref[...], preferred_element_type=jnp.float32) + bq_ref[...]
  k_ = jnp.dot(comp, wk_ref[...], preferred_element_type=jnp.float32) + bk_ref[...]
  v_ = jnp.dot(comp, wv_ref[...], preferred_element_type=jnp.float32) + bv_ref[...]

  dh = D // H
  scale = 1.0 / (dh ** 0.5)
  outs = []
  for hh in range(H):
    sl = slice(hh * dh, (hh + 1) * dh)
    sc = (k_[:, sl] * q[:, sl]).sum(-1, keepdims=True) * scale
    sc = jnp.where(cv, sc, -1e9)
    mx = sc.max(0, keepdims=True)
    e = jnp.exp(sc - mx)
    a = e / e.sum(0, keepdims=True)
    outs.append((a * v_[:, sl]).sum(0, keepdims=True))
  o = jnp.concatenate(outs, axis=1)
  att = jnp.dot(o, wo_ref[...], preferred_element_type=jnp.float32) + bo_ref[...]
  out_ref[0] = jnp.dot(att, cw_ref[...],
                       preferred_element_type=jnp.float32) + cb_ref[...]


def _tail(rows0, rows1, sv0, sv1, temb3, enc_W, enc_b, ln_g, ln_b,
          Wq, bq, Wk, bk, Wv, bv, Wo, bo, cls_W, cls_b, H):
  B, L, FD = rows0.shape
  D = enc_W.shape[1]
  NC = cls_W.shape[1]
  rspec = pl.BlockSpec((1, L, FD), lambda b: (b, 0, 0))
  svspec = pl.BlockSpec((1, L, 1), lambda b: (b, 0, 0))
  wspec = pl.BlockSpec((D, D), lambda b: (0, 0))
  bspec = pl.BlockSpec((1, D), lambda b: (0, 0))
  out = pl.pallas_call(
      functools.partial(_tail_body, H),
      grid=(B,),
      in_specs=[
          rspec, rspec, svspec, svspec,
          pl.BlockSpec((1, 1, D), lambda b: (b, 0, 0)),
          pl.BlockSpec((FD, D), lambda b: (0, 0)), bspec, bspec, bspec,
          wspec, bspec, wspec, bspec, wspec, bspec, wspec, bspec,
          pl.BlockSpec((D, NC), lambda b: (0, 0)),
          pl.BlockSpec((1, NC), lambda b: (0, 0)),
      ],
      out_specs=pl.BlockSpec((1, 1, NC), lambda b: (b, 0, 0)),
      out_shape=jax.ShapeDtypeStruct((B, 1, NC), jnp.float32),
  )(rows0, rows1, sv0, sv1, temb3,
    enc_W, enc_b.reshape(1, D), ln_g.reshape(1, D), ln_b.reshape(1, D),
    Wq, bq.reshape(1, D), Wk, bk.reshape(1, D), Wv, bv.reshape(1, D),
    Wo, bo.reshape(1, D), cls_W, cls_b.reshape(1, NC))
  return out.reshape(B, NC)


def kernel(padded_bags, key_padding_mask, input_ids, attention_mask,
           text_emb_table, text_pool_W, text_pool_b,
           text_proj_W, text_proj_b, enc_W, enc_b, ln_g, ln_b,
           Wq, bq, Wk, bk, Wv, bv, Wo, bo, cls_W, cls_b):
  B, N, FD = padded_bags.shape
  S = input_ids.shape[1]
  D = enc_W.shape[1]
  H = 8
  kk = min(L_MAX, max(1, int(GAMMA * N)))

  tok = _sc_gather(text_emb_table, input_ids.reshape(B * S), 16)
  tok = tok.reshape(B, S, -1)

  temb = _text_emb(tok, attention_mask.astype(jnp.float32),
                   text_pool_W, text_pool_b, text_proj_W, text_proj_b)
  temb3 = temb.reshape(B, 1, D)

  kpm_f = key_padding_mask.astype(jnp.float32)
  rel, val = _encode_rel(padded_bags, kpm_f, temb3, enc_W, enc_b, ln_g, ln_b)

  gidx0, sv0, gidx1, sv1 = _select(rel, val, kk)
  L = kk // 2

  bags_flat = padded_bags.reshape(B * N, FD)
  rows0 = _sc_gather(bags_flat, gidx0.reshape(B * L), 128).reshape(B, L, FD)
  rows1 = _sc_gather(bags_flat, gidx1.reshape(B * L), 128).reshape(B, L, FD)

  return _tail(rows0, rows1, sv0, sv1, temb3, enc_W, enc_b, ln_g, ln_b,
               Wq, bq, Wk, bk, Wv, bv, Wo, bo, cls_W, cls_b, H)


# speedup vs baseline: 1.5299x; 1.2400x over previous
import functools

import jax
import jax.numpy as jnp
from jax import lax
from jax.experimental import pallas as pl
from jax.experimental.pallas import tpu as pltpu
from jax.experimental.pallas import tpu_sc as plsc

THR = 0.7
GAMMA = 0.8
L_MAX = 1024
TN = 2048


def _sc_gather(table, idx, rows_per_chunk):
  M = idx.shape[0]
  D = table.shape[1]
  info = plsc.get_sparse_core_info()
  nw = info.num_cores * info.num_subcores
  b_per_w = M // nw
  nchunks = b_per_w // rows_per_chunk
  mesh = plsc.VectorSubcoreMesh(core_axis_name="c", subcore_axis_name="s")

  @functools.partial(
      pl.kernel,
      mesh=mesh,
      out_type=jax.ShapeDtypeStruct((M, D), table.dtype),
      scratch_types=[
          pltpu.VMEM((rows_per_chunk,), jnp.int32),
          pltpu.VMEM((rows_per_chunk, D), table.dtype),
          pltpu.SemaphoreType.DMA,
      ],
  )
  def gk(table_hbm, idx_hbm, out_hbm, idx_v, rows_v, sem):
    wid = lax.axis_index("s") * info.num_cores + lax.axis_index("c")
    base = wid * b_per_w
    for c in range(nchunks):
      off = base + c * rows_per_chunk
      pltpu.sync_copy(idx_hbm.at[pl.ds(off, rows_per_chunk)], idx_v)
      pltpu.async_copy(table_hbm.at[idx_v], rows_v, sem).wait()
      pltpu.sync_copy(rows_v, out_hbm.at[pl.ds(off, rows_per_chunk)])

  return gk(table, idx)


def _text_body(tok_ref, am_ref, pw_ref, pb_ref, qw_ref, qb_ref, out_ref):
  tok = tok_ref[...]
  m = am_ref[...]
  pooled = (tok * m[..., None]).sum(1) / jnp.maximum(m.sum(1, keepdims=True), 1.0)
  feat = jnp.tanh(jnp.dot(pooled, pw_ref[...],
                          preferred_element_type=jnp.float32) + pb_ref[...])
  out_ref[...] = jnp.dot(feat, qw_ref[...],
                         preferred_element_type=jnp.float32) + qb_ref[...]


def _text_emb(tok, am_f, pool_W, pool_b, proj_W, proj_b):
  B = tok.shape[0]
  D = proj_W.shape[1]
  return pl.pallas_call(
      _text_body,
      out_shape=jax.ShapeDtypeStruct((B, D), jnp.float32),
  )(tok, am_f, pool_W, pool_b.reshape(1, -1), proj_W, proj_b.reshape(1, -1))


def _enc_tile(x, w, b, g, bb):
  h = jnp.dot(x, w, preferred_element_type=jnp.float32) + b
  mu = h.mean(-1, keepdims=True)
  var = ((h - mu) ** 2).mean(-1, keepdims=True)
  h = (h - mu) / jnp.sqrt(var + 1e-5) * g + bb
  return jnp.maximum(h, 0.0)


def _enc_body(bags_ref, kpm_ref, temb_ref, w_ref, b_ref, g_ref, bb_ref,
              rel_ref, val_ref, carry_ref):
  j = pl.program_id(1)
  img = _enc_tile(bags_ref[0], w_ref[...], b_ref[...], g_ref[...], bb_ref[...])
  t = temb_ref[0]
  rel = jnp.dot(img, t.reshape(-1, 1),
                preferred_element_type=jnp.float32)
  nrm = img / (jnp.sqrt((img * img).sum(-1, keepdims=True)) + 1e-8)
  prev = jnp.concatenate([carry_ref[0:1], nrm[:-1]], axis=0)
  sim = (nrm * prev).sum(-1, keepdims=True)
  carry_ref[0:1] = nrm[-1:]
  red = (sim > THR).astype(jnp.float32)
  row = lax.broadcasted_iota(jnp.int32, sim.shape, 0)
  red = jnp.where((j == 0) & (row == 0), 0.0, red)
  valid = (1.0 - kpm_ref[0]) * (1.0 - red)
  rel_ref[0] = jnp.where(valid > 0.0, rel, -1e9)
  val_ref[0] = valid


def _encode_rel(bags, kpm_f, temb3, enc_W, enc_b, ln_g, ln_b):
  B, N, FD = bags.shape
  D = enc_W.shape[1]
  nt = N // TN
  grid = (B, nt)
  out = pl.pallas_call(
      _enc_body,
      grid=grid,
      in_specs=[
          pl.BlockSpec((1, TN, FD), lambda b, j: (b, j, 0)),
          pl.BlockSpec((1, TN, 1), lambda b, j: (b * nt + j, 0, 0)),
          pl.BlockSpec((1, 1, D), lambda b, j: (b, 0, 0)),
          pl.BlockSpec((FD, D), lambda b, j: (0, 0)),
          pl.BlockSpec((1, D), lambda b, j: (0, 0)),
          pl.BlockSpec((1, D), lambda b, j: (0, 0)),
          pl.BlockSpec((1, D), lambda b, j: (0, 0)),
      ],
      out_specs=[
          pl.BlockSpec((1, TN, 1), lambda b, j: (b * nt + j, 0, 0)),
          pl.BlockSpec((1, TN, 1), lambda b, j: (b * nt + j, 0, 0)),
      ],
      out_shape=[
          jax.ShapeDtypeStruct((B * nt, TN, 1), jnp.float32),
          jax.ShapeDtypeStruct((B * nt, TN, 1), jnp.float32),
      ],
      scratch_shapes=[pltpu.VMEM((8, D), jnp.float32)],
      compiler_params=pltpu.CompilerParams(
          dimension_semantics=("arbitrary", "arbitrary")),
  )(bags, kpm_f.reshape(B * nt, TN, 1), temb3,
    enc_W, enc_b.reshape(1, D), ln_g.reshape(1, D), ln_b.reshape(1, D))
  return out[0].reshape(B, N), out[1].reshape(B, N)


def _sel_body(kk, N, rel_ref, val_ref, gidx0_ref, sv0_ref, gidx1_ref, sv1_ref):
  b = pl.program_id(0)
  x = rel_ref[0]
  R, C = x.shape
  bits = lax.bitcast_convert_type(x, jnp.int32)
  key = jnp.where(bits >= 0, bits, bits ^ jnp.int32(2147483647))

  cnt_pos = jnp.sum((key >= 0).astype(jnp.int32))
  t0 = jnp.where(cnt_pos >= kk, jnp.int32(0), jnp.iinfo(jnp.int32).min)

  def bit_step(i, t):
    cand = t | (jnp.int32(1) << (jnp.int32(30) - i))
    cnt = jnp.sum((key >= cand).astype(jnp.int32))
    return jnp.where(cnt >= kk, cand, t)

  t = lax.fori_loop(0, 31, bit_step, t0)

  gt = (key > t).astype(jnp.float32)
  eq = (key == t).astype(jnp.float32)
  need = kk - jnp.sum(gt)

  ri = lax.broadcasted_iota(jnp.int32, (R, C), 0)
  ci = lax.broadcasted_iota(jnp.int32, (R, C), 1)
  m_inc = (ri <= ci).astype(jnp.float32)
  m_gt = (ri > ci).astype(jnp.float32)
  m_ge = (ri >= ci).astype(jnp.float32)

  incl_eq = jnp.dot(eq, m_inc)
  rowsum_eq = incl_eq[:, C - 1:C]
  rowpref_eq = jnp.dot(m_gt, rowsum_eq)
  excl_eq = rowpref_eq + incl_eq - eq
  take = gt + eq * (excl_eq < need).astype(jnp.float32)

  incl_tk = jnp.dot(take, m_inc)
  rowsum_tk = incl_tk[:, C - 1:C]
  rowincl_tk = jnp.dot(m_ge, rowsum_tk)
  rowpref_col = rowincl_tk - rowsum_tk

  half = kk // 2
  lane = lax.broadcasted_iota(jnp.int32, (1, half), 1).astype(jnp.float32)
  rowiota_c = lax.broadcasted_iota(jnp.int32, (R, 1), 0).astype(jnp.float32)

  def extract(off):
    jj = lane * 2.0 + off
    r_j = (rowincl_tk <= jj).astype(jnp.float32).sum(
        0, keepdims=True)
    onehot_r = (rowiota_c == r_j).astype(jnp.float32)
    rowpref_j = (onehot_r * rowpref_col).sum(0, keepdims=True)
    jl = jj - rowpref_j
    inclT = lax.dot_general(incl_tk, onehot_r,
                            (((0,), (0,)), ((), ())))
    c_j = (inclT <= jl).astype(jnp.float32).sum(0, keepdims=True)
    onehot_c = (rowiota_c == c_j).astype(jnp.float32)
    vT = lax.dot_general(val_ref[0], onehot_r,
                         (((0,), (0,)), ((), ())))
    sv = (vT * onehot_c).sum(0, keepdims=True)
    gidx = (r_j * C + c_j).astype(jnp.int32) + b * N
    return gidx, sv

  g0, s0 = extract(0)
  g1, s1 = extract(1)
  gidx0_ref[0] = g0
  sv0_ref[0] = s0
  gidx1_ref[0] = g1
  sv1_ref[0] = s1


def _select(rel, val, kk):
  B, N = rel.shape
  R = 128
  C = N // R
  half = kk // 2
  ospec = pl.BlockSpec((1, 1, half), lambda b: (b, 0, 0))
  return pl.pallas_call(
      functools.partial(_sel_body, kk, N),
      grid=(B,),
      in_specs=[
          pl.BlockSpec((1, R, C), lambda b: (b, 0, 0)),
          pl.BlockSpec((1, R, C), lambda b: (b, 0, 0)),
      ],
      out_specs=[ospec, ospec, ospec, ospec],
      out_shape=[
          jax.ShapeDtypeStruct((B, 1, half), jnp.int32),
          jax.ShapeDtypeStruct((B, 1, half), jnp.float32),
          jax.ShapeDtypeStruct((B, 1, half), jnp.int32),
          jax.ShapeDtypeStruct((B, 1, half), jnp.float32),
      ],
  )(rel.reshape(B, R, C), val.reshape(B, R, C))


def _tail_body(H, rows0_ref, rows1_ref, sv0_ref, sv1_ref, temb_ref,
               w_ref, b_ref, g_ref, bb_ref,
               wq_ref, bq_ref, wk_ref, bk_ref, wv_ref, bv_ref,
               wo_ref, bo_ref, cw_ref, cb_ref, out_ref):
  f0 = _enc_tile(rows0_ref[0], w_ref[...], b_ref[...], g_ref[...],
                 bb_ref[...])
  f1 = _enc_tile(rows1_ref[0], w_ref[...], b_ref[...], g_ref[...],
                 bb_ref[...])
  L, D = f0.shape
  v0 = sv0_ref[0]
  v1 = sv1_ref[0]
  comp = (f0 * v0 + f1 * v1) / jnp.maximum(v0 + v1, 1.0)
  cv = (v0 + v1) > 0.0

  t = temb_ref[0]
  q = jnp.dot(t, wq_ref[...], preferred_element_type=jnp.float32) + bq_ref[...]
  k_ = jnp.dot(comp, wk_ref[...], preferred_element_type=jnp.float32) + bk_ref[...]
  v_ = jnp.dot(comp, wv_ref[...], preferred_element_type=jnp.float32) + bv_ref[...]

  dh = D // H
  scale = 1.0 / (dh ** 0.5)
  outs = []
  for hh in range(H):
    sl = slice(hh * dh, (hh + 1) * dh)
    sc = (k_[:, sl] * q[:, sl]).sum(-1, keepdims=True) * scale
    sc = jnp.where(cv, sc, -1e9)
    mx = sc.max(0, keepdims=True)
    e = jnp.exp(sc - mx)
    a = e / e.sum(0, keepdims=True)
    outs.append((a * v_[:, sl]).sum(0, keepdims=True))
  o = jnp.concatenate(outs, axis=1)
  att = jnp.dot(o, wo_ref[...], preferred_element_type=jnp.float32) + bo_ref[...]
  out_ref[0] = jnp.dot(att, cw_ref[...],
                       preferred_element_type=jnp.float32) + cb_ref[...]


def _tail(rows0, rows1, sv0, sv1, temb3, enc_W, enc_b, ln_g, ln_b,
          Wq, bq, Wk, bk, Wv, bv, Wo, bo, cls_W, cls_b, H):
  B, L, FD = rows0.shape
  D = enc_W.shape[1]
  NC = cls_W.shape[1]
  rspec = pl.BlockSpec((1, L, FD), lambda b: (b, 0, 0))
  svspec = pl.BlockSpec((1, L, 1), lambda b: (b, 0, 0))
  wspec = pl.BlockSpec((D, D), lambda b: (0, 0))
  bspec = pl.BlockSpec((1, D), lambda b: (0, 0))
  out = pl.pallas_call(
      functools.partial(_tail_body, H),
      grid=(B,),
      in_specs=[
          rspec, rspec, svspec, svspec,
          pl.BlockSpec((1, 1, D), lambda b: (b, 0, 0)),
          pl.BlockSpec((FD, D), lambda b: (0, 0)), bspec, bspec, bspec,
          wspec, bspec, wspec, bspec, wspec, bspec, wspec, bspec,
          pl.BlockSpec((D, NC), lambda b: (0, 0)),
          pl.BlockSpec((1, NC), lambda b: (0, 0)),
      ],
      out_specs=pl.BlockSpec((1, 1, NC), lambda b: (b, 0, 0)),
      out_shape=jax.ShapeDtypeStruct((B, 1, NC), jnp.float32),
  )(rows0, rows1, sv0, sv1, temb3,
    enc_W, enc_b.reshape(1, D), ln_g.reshape(1, D), ln_b.reshape(1, D),
    Wq, bq.reshape(1, D), Wk, bk.reshape(1, D), Wv, bv.reshape(1, D),
    Wo, bo.reshape(1, D), cls_W, cls_b.reshape(1, NC))
  return out.reshape(B, NC)


def kernel(padded_bags, key_padding_mask, input_ids, attention_mask,
           text_emb_table, text_pool_W, text_pool_b,
           text_proj_W, text_proj_b, enc_W, enc_b, ln_g, ln_b,
           Wq, bq, Wk, bk, Wv, bv, Wo, bo, cls_W, cls_b):
  B, N, FD = padded_bags.shape
  S = input_ids.shape[1]
  D = enc_W.shape[1]
  H = 8
  kk = min(L_MAX, max(1, int(GAMMA * N)))

  tok = _sc_gather(text_emb_table, input_ids.reshape(B * S), 16)
  tok = tok.reshape(B, S, -1)

  temb = _text_emb(tok, attention_mask.astype(jnp.float32),
                   text_pool_W, text_pool_b, text_proj_W, text_proj_b)
  temb3 = temb.reshape(B, 1, D)

  kpm_f = key_padding_mask.astype(jnp.float32)
  rel, val = _encode_rel(padded_bags, kpm_f, temb3, enc_W, enc_b, ln_g, ln_b)

  gidx0, sv0, gidx1, sv1 = _select(rel, val, kk)
  L = kk // 2

  bags_flat = padded_bags.reshape(B * N, FD)
  gidx_all = jnp.concatenate(
      [gidx0.reshape(B * L), gidx1.reshape(B * L)], axis=0)
  rows_all = _sc_gather(bags_flat, gidx_all, 128)
  rows0 = rows_all[:B * L].reshape(B, L, FD)
  rows1 = rows_all[B * L:].reshape(B, L, FD)
  sv0 = sv0.reshape(B, L, 1)
  sv1 = sv1.reshape(B, L, 1)

  return _tail(rows0, rows1, sv0, sv1, temb3, enc_W, enc_b, ln_g, ln_b,
               Wq, bq, Wk, bk, Wv, bv, Wo, bo, cls_W, cls_b, H)


# speedup vs baseline: 1.6014x; 1.0467x over previous
import functools

import jax
import jax.numpy as jnp
from jax import lax
from jax.experimental import pallas as pl
from jax.experimental.pallas import tpu as pltpu
from jax.experimental.pallas import tpu_sc as plsc

THR = 0.7
GAMMA = 0.8
L_MAX = 1024
TN = 4096


def _sc_gather(table, idx, rows_per_chunk):
  M = idx.shape[0]
  D = table.shape[1]
  info = plsc.get_sparse_core_info()
  nw = info.num_cores * info.num_subcores
  b_per_w = M // nw
  nchunks = b_per_w // rows_per_chunk
  mesh = plsc.VectorSubcoreMesh(core_axis_name="c", subcore_axis_name="s")

  @functools.partial(
      pl.kernel,
      mesh=mesh,
      out_type=jax.ShapeDtypeStruct((M, D), table.dtype),
      scratch_types=[
          pltpu.VMEM((rows_per_chunk,), jnp.int32),
          pltpu.VMEM((rows_per_chunk, D), table.dtype),
          pltpu.SemaphoreType.DMA,
      ],
  )
  def gk(table_hbm, idx_hbm, out_hbm, idx_v, rows_v, sem):
    wid = lax.axis_index("s") * info.num_cores + lax.axis_index("c")
    base = wid * b_per_w
    for c in range(nchunks):
      off = base + c * rows_per_chunk
      pltpu.sync_copy(idx_hbm.at[pl.ds(off, rows_per_chunk)], idx_v)
      pltpu.async_copy(table_hbm.at[idx_v], rows_v, sem).wait()
      pltpu.sync_copy(rows_v, out_hbm.at[pl.ds(off, rows_per_chunk)])

  return gk(table, idx)


def _text_body(tok_ref, am_ref, pw_ref, pb_ref, qw_ref, qb_ref, out_ref):
  tok = tok_ref[...]
  m = am_ref[...]
  pooled = (tok * m[..., None]).sum(1) / jnp.maximum(m.sum(1, keepdims=True), 1.0)
  feat = jnp.tanh(jnp.dot(pooled, pw_ref[...],
                          preferred_element_type=jnp.float32) + pb_ref[...])
  out_ref[...] = jnp.dot(feat, qw_ref[...],
                         preferred_element_type=jnp.float32) + qb_ref[...]


def _text_emb(tok, am_f, pool_W, pool_b, proj_W, proj_b):
  B = tok.shape[0]
  D = proj_W.shape[1]
  return pl.pallas_call(
      _text_body,
      out_shape=jax.ShapeDtypeStruct((B, D), jnp.float32),
  )(tok, am_f, pool_W, pool_b.reshape(1, -1), proj_W, proj_b.reshape(1, -1))


def _enc_tile(x, w, b, g, bb):
  h = jnp.dot(x, w, preferred_element_type=jnp.float32) + b
  mu = h.mean(-1, keepdims=True)
  var = ((h - mu) ** 2).mean(-1, keepdims=True)
  h = (h - mu) / jnp.sqrt(var + 1e-5) * g + bb
  return jnp.maximum(h, 0.0)


def _enc_body(bags_ref, kpm_ref, temb_ref, w_ref, b_ref, g_ref, bb_ref,
              rel_ref, val_ref, carry_ref):
  j = pl.program_id(1)
  img = _enc_tile(bags_ref[0], w_ref[...], b_ref[...], g_ref[...], bb_ref[...])
  t = temb_ref[0]
  rel = jnp.dot(img, t.reshape(-1, 1),
                preferred_element_type=jnp.float32)
  nrm = img / (jnp.sqrt((img * img).sum(-1, keepdims=True)) + 1e-8)
  prev = jnp.concatenate([carry_ref[0:1], nrm[:-1]], axis=0)
  sim = (nrm * prev).sum(-1, keepdims=True)
  carry_ref[0:1] = nrm[-1:]
  red = (sim > THR).astype(jnp.float32)
  row = lax.broadcasted_iota(jnp.int32, sim.shape, 0)
  red = jnp.where((j == 0) & (row == 0), 0.0, red)
  valid = (1.0 - kpm_ref[0]) * (1.0 - red)
  rel_ref[0] = jnp.where(valid > 0.0, rel, -1e9)
  val_ref[0] = valid


def _encode_rel(bags, kpm_f, temb3, enc_W, enc_b, ln_g, ln_b):
  B, N, FD = bags.shape
  D = enc_W.shape[1]
  nt = N // TN
  grid = (B, nt)
  out = pl.pallas_call(
      _enc_body,
      grid=grid,
      in_specs=[
          pl.BlockSpec((1, TN, FD), lambda b, j: (b, j, 0)),
          pl.BlockSpec((1, TN, 1), lambda b, j: (b * nt + j, 0, 0)),
          pl.BlockSpec((1, 1, D), lambda b, j: (b, 0, 0)),
          pl.BlockSpec((FD, D), lambda b, j: (0, 0)),
          pl.BlockSpec((1, D), lambda b, j: (0, 0)),
          pl.BlockSpec((1, D), lambda b, j: (0, 0)),
          pl.BlockSpec((1, D), lambda b, j: (0, 0)),
      ],
      out_specs=[
          pl.BlockSpec((1, TN, 1), lambda b, j: (b * nt + j, 0, 0)),
          pl.BlockSpec((1, TN, 1), lambda b, j: (b * nt + j, 0, 0)),
      ],
      out_shape=[
          jax.ShapeDtypeStruct((B * nt, TN, 1), jnp.float32),
          jax.ShapeDtypeStruct((B * nt, TN, 1), jnp.float32),
      ],
      scratch_shapes=[pltpu.VMEM((8, D), jnp.float32)],
      compiler_params=pltpu.CompilerParams(
          dimension_semantics=("arbitrary", "arbitrary")),
  )(bags, kpm_f.reshape(B * nt, TN, 1), temb3,
    enc_W, enc_b.reshape(1, D), ln_g.reshape(1, D), ln_b.reshape(1, D))
  return out[0].reshape(B, N), out[1].reshape(B, N)


def _sel_body(kk, N, rel_ref, val_ref, gidx0_ref, sv0_ref, gidx1_ref, sv1_ref):
  b = pl.program_id(0)
  x = rel_ref[0]
  R, C = x.shape
  bits = lax.bitcast_convert_type(x, jnp.int32)
  key = jnp.where(bits >= 0, bits, bits ^ jnp.int32(2147483647))

  cnt_pos = jnp.sum((key >= 0).astype(jnp.int32))
  t0 = jnp.where(cnt_pos >= kk, jnp.int32(0), jnp.iinfo(jnp.int32).min)

  def bit_step(i, t):
    cand = t | (jnp.int32(1) << (jnp.int32(30) - i))
    cnt = jnp.sum((key >= cand).astype(jnp.int32))
    return jnp.where(cnt >= kk, cand, t)

  t = lax.fori_loop(0, 31, bit_step, t0)

  gt = (key > t).astype(jnp.float32)
  eq = (key == t).astype(jnp.float32)
  need = kk - jnp.sum(gt)

  ri = lax.broadcasted_iota(jnp.int32, (R, C), 0)
  ci = lax.broadcasted_iota(jnp.int32, (R, C), 1)
  m_inc = (ri <= ci).astype(jnp.float32)
  m_gt = (ri > ci).astype(jnp.float32)
  m_ge = (ri >= ci).astype(jnp.float32)

  incl_eq = jnp.dot(eq, m_inc)
  rowsum_eq = incl_eq[:, C - 1:C]
  rowpref_eq = jnp.dot(m_gt, rowsum_eq)
  excl_eq = rowpref_eq + incl_eq - eq
  take = gt + eq * (excl_eq < need).astype(jnp.float32)

  incl_tk = jnp.dot(take, m_inc)
  rowsum_tk = incl_tk[:, C - 1:C]
  rowincl_tk = jnp.dot(m_ge, rowsum_tk)
  rowpref_col = rowincl_tk - rowsum_tk

  half = kk // 2
  lane = lax.broadcasted_iota(jnp.int32, (1, half), 1).astype(jnp.float32)
  rowiota_c = lax.broadcasted_iota(jnp.int32, (R, 1), 0).astype(jnp.float32)

  def extract(off):
    jj = lane * 2.0 + off
    r_j = (rowincl_tk <= jj).astype(jnp.float32).sum(
        0, keepdims=True)
    onehot_r = (rowiota_c == r_j).astype(jnp.float32)
    rowpref_j = (onehot_r * rowpref_col).sum(0, keepdims=True)
    jl = jj - rowpref_j
    inclT = lax.dot_general(incl_tk, onehot_r,
                            (((0,), (0,)), ((), ())))
    c_j = (inclT <= jl).astype(jnp.float32).sum(0, keepdims=True)
    onehot_c = (rowiota_c == c_j).astype(jnp.float32)
    vT = lax.dot_general(val_ref[0], onehot_r,
                         (((0,), (0,)), ((), ())))
    sv = (vT * onehot_c).sum(0, keepdims=True)
    gidx = (r_j * C + c_j).astype(jnp.int32) + b * N
    return gidx, sv

  g0, s0 = extract(0)
  g1, s1 = extract(1)
  gidx0_ref[0] = g0
  sv0_ref[0] = s0
  gidx1_ref[0] = g1
  sv1_ref[0] = s1


def _select(rel, val, kk):
  B, N = rel.shape
  R = 128
  C = N // R
  half = kk // 2
  ospec = pl.BlockSpec((1, 1, half), lambda b: (b, 0, 0))
  return pl.pallas_call(
      functools.partial(_sel_body, kk, N),
      grid=(B,),
      in_specs=[
          pl.BlockSpec((1, R, C), lambda b: (b, 0, 0)),
          pl.BlockSpec((1, R, C), lambda b: (b, 0, 0)),
      ],
      out_specs=[ospec, ospec, ospec, ospec],
      out_shape=[
          jax.ShapeDtypeStruct((B, 1, half), jnp.int32),
          jax.ShapeDtypeStruct((B, 1, half), jnp.float32),
          jax.ShapeDtypeStruct((B, 1, half), jnp.int32),
          jax.ShapeDtypeStruct((B, 1, half), jnp.float32),
      ],
  )(rel.reshape(B, R, C), val.reshape(B, R, C))


def _tail_body(H, rows0_ref, rows1_ref, sv0_ref, sv1_ref, temb_ref,
               w_ref, b_ref, g_ref, bb_ref,
               wq_ref, bq_ref, wk_ref, bk_ref, wv_ref, bv_ref,
               wo_ref, bo_ref, cw_ref, cb_ref, out_ref):
  f0 = _enc_tile(rows0_ref[0], w_ref[...], b_ref[...], g_ref[...],
                 bb_ref[...])
  f1 = _enc_tile(rows1_ref[0], w_ref[...], b_ref[...], g_ref[...],
                 bb_ref[...])
  L, D = f0.shape
  v0 = sv0_ref[0]
  v1 = sv1_ref[0]
  comp = (f0 * v0 + f1 * v1) / jnp.maximum(v0 + v1, 1.0)
  cv = (v0 + v1) > 0.0

  t = temb_ref[0]
  q = jnp.dot(t, wq_ref[...], preferred_element_type=jnp.float32) + bq_ref[...]
  k_ = jnp.dot(comp, wk_ref[...], preferred_element_type=jnp.float32) + bk_ref[...]
  v_ = jnp.dot(comp, wv_ref[...], preferred_element_type=jnp.float32) + bv_ref[...]

  dh = D // H
  scale = 1.0 / (dh ** 0.5)
  outs = []
  for hh in range(H):
    sl = slice(hh * dh, (hh + 1) * dh)
    sc = (k_[:, sl] * q[:, sl]).sum(-1, keepdims=True) * scale
    sc = jnp.where(cv, sc, -1e9)
    mx = sc.max(0, keepdims=True)
    e = jnp.exp(sc - mx)
    a = e / e.sum(0, keepdims=True)
    outs.append((a * v_[:, sl]).sum(0, keepdims=True))
  o = jnp.concatenate(outs, axis=1)
  att = jnp.dot(o, wo_ref[...], preferred_element_type=jnp.float32) + bo_ref[...]
  out_ref[0] = jnp.dot(att, cw_ref[...],
                       preferred_element_type=jnp.float32) + cb_ref[...]


def _tail(rows0, rows1, sv0, sv1, temb3, enc_W, enc_b, ln_g, ln_b,
          Wq, bq, Wk, bk, Wv, bv, Wo, bo, cls_W, cls_b, H):
  B, L, FD = rows0.shape
  D = enc_W.shape[1]
  NC = cls_W.shape[1]
  rspec = pl.BlockSpec((1, L, FD), lambda b: (b, 0, 0))
  svspec = pl.BlockSpec((1, L, 1), lambda b: (b, 0, 0))
  wspec = pl.BlockSpec((D, D), lambda b: (0, 0))
  bspec = pl.BlockSpec((1, D), lambda b: (0, 0))
  out = pl.pallas_call(
      functools.partial(_tail_body, H),
      grid=(B,),
      in_specs=[
          rspec, rspec, svspec, svspec,
          pl.BlockSpec((1, 1, D), lambda b: (b, 0, 0)),
          pl.BlockSpec((FD, D), lambda b: (0, 0)), bspec, bspec, bspec,
          wspec, bspec, wspec, bspec, wspec, bspec, wspec, bspec,
          pl.BlockSpec((D, NC), lambda b: (0, 0)),
          pl.BlockSpec((1, NC), lambda b: (0, 0)),
      ],
      out_specs=pl.BlockSpec((1, 1, NC), lambda b: (b, 0, 0)),
      out_shape=jax.ShapeDtypeStruct((B, 1, NC), jnp.float32),
  )(rows0, rows1, sv0, sv1, temb3,
    enc_W, enc_b.reshape(1, D), ln_g.reshape(1, D), ln_b.reshape(1, D),
    Wq, bq.reshape(1, D), Wk, bk.reshape(1, D), Wv, bv.reshape(1, D),
    Wo, bo.reshape(1, D), cls_W, cls_b.reshape(1, NC))
  return out.reshape(B, NC)


def kernel(padded_bags, key_padding_mask, input_ids, attention_mask,
           text_emb_table, text_pool_W, text_pool_b,
           text_proj_W, text_proj_b, enc_W, enc_b, ln_g, ln_b,
           Wq, bq, Wk, bk, Wv, bv, Wo, bo, cls_W, cls_b):
  B, N, FD = padded_bags.shape
  S = input_ids.shape[1]
  D = enc_W.shape[1]
  H = 8
  kk = min(L_MAX, max(1, int(GAMMA * N)))

  tok = _sc_gather(text_emb_table, input_ids.reshape(B * S), 16)
  tok = tok.reshape(B, S, -1)

  temb = _text_emb(tok, attention_mask.astype(jnp.float32),
                   text_pool_W, text_pool_b, text_proj_W, text_proj_b)
  temb3 = temb.reshape(B, 1, D)

  kpm_f = key_padding_mask.astype(jnp.float32)
  rel, val = _encode_rel(padded_bags, kpm_f, temb3, enc_W, enc_b, ln_g, ln_b)

  gidx0, sv0, gidx1, sv1 = _select(rel, val, kk)
  L = kk // 2

  bags_flat = padded_bags.reshape(B * N, FD)
  gidx_all = jnp.concatenate(
      [gidx0.reshape(B * L), gidx1.reshape(B * L)], axis=0)
  rows_all = _sc_gather(bags_flat, gidx_all, 128)
  rows0 = rows_all[:B * L].reshape(B, L, FD)
  rows1 = rows_all[B * L:].reshape(B, L, FD)
  sv0 = sv0.reshape(B, L, 1)
  sv1 = sv1.reshape(B, L, 1)

  return _tail(rows0, rows1, sv0, sv1, temb3, enc_W, enc_b, ln_g, ln_b,
               Wq, bq, Wk, bk, Wv, bv, Wo, bo, cls_W, cls_b, H)
